# speedup vs baseline: 2.3642x; 2.3642x over previous
import jax
import jax.numpy as jnp
from jax.experimental import pallas as pl

N1 = 128
D_HALF = 512
BS = 8


def _tc_body(x_ref, p1_ref, p2_ref, o_ref):
    x = x_ref[...]
    p1 = p1_ref[...]
    p2 = p2_ref[...]
    lo = x[:, :, :D_HALF] + p1[None, :, :]
    hi = x[:, :, D_HALF:] + p2[:, None, :]
    o_ref[...] = jnp.concatenate([lo, hi], axis=-1)


@jax.jit
def kernel(x, params1, params2):
    b, num_tokens, d_in = x.shape
    n_sblocks = num_tokens // N1
    rows = b * n_sblocks
    x4 = x.reshape(rows, N1, d_in)
    grid = (rows // BS,)
    out = pl.pallas_call(
        _tc_body,
        grid=grid,
        in_specs=[
            pl.BlockSpec((BS, N1, d_in), lambda j: (j, 0, 0)),
            pl.BlockSpec((N1, D_HALF), lambda j: (0, 0)),
            pl.BlockSpec((BS, D_HALF), lambda j: (j % (n_sblocks // BS), 0)),
        ],
        out_specs=pl.BlockSpec((BS, N1, d_in), lambda j: (j, 0, 0)),
        out_shape=jax.ShapeDtypeStruct((rows, N1, d_in), x.dtype),
    )(x4, params1, params2)
    return out.reshape(b, num_tokens, d_in)


# speedup vs baseline: 2.3668x; 1.0011x over previous
import jax
import jax.numpy as jnp
from jax.experimental import pallas as pl

N1 = 128
D_HALF = 512
BS = 8


def _tc_body(x_ref, p1_ref, p2_ref, o_ref):
    x = x_ref[...]
    p1 = p1_ref[...]
    p2 = p2_ref[...]
    o_ref[:, :, :D_HALF] = x[:, :, :D_HALF] + p1[None, :, :]
    o_ref[:, :, D_HALF:] = x[:, :, D_HALF:] + p2[:, None, :]


@jax.jit
def kernel(x, params1, params2):
    b, num_tokens, d_in = x.shape
    n_sblocks = num_tokens // N1
    rows = b * n_sblocks
    x4 = x.reshape(rows, N1, d_in)
    grid = (rows // BS,)
    out = pl.pallas_call(
        _tc_body,
        grid=grid,
        in_specs=[
            pl.BlockSpec((BS, N1, d_in), lambda j: (j, 0, 0)),
            pl.BlockSpec((N1, D_HALF), lambda j: (0, 0)),
            pl.BlockSpec((BS, D_HALF), lambda j: (j % (n_sblocks // BS), 0)),
        ],
        out_specs=pl.BlockSpec((BS, N1, d_in), lambda j: (j, 0, 0)),
        out_shape=jax.ShapeDtypeStruct((rows, N1, d_in), x.dtype),
    )(x4, params1, params2)
    return out.reshape(b, num_tokens, d_in)


# speedup vs baseline: 2.3989x; 1.0136x over previous
import jax
import jax.numpy as jnp
from jax.experimental import pallas as pl

N1 = 128
D_HALF = 512
BS = 16


def _tc_body(x_ref, p1_ref, p2_ref, o_ref):
    x = x_ref[...]
    p1 = p1_ref[...]
    p2 = p2_ref[...]
    o_ref[:, :, :D_HALF] = x[:, :, :D_HALF] + p1[None, :, :]
    o_ref[:, :, D_HALF:] = x[:, :, D_HALF:] + p2[:, None, :]


@jax.jit
def kernel(x, params1, params2):
    b, num_tokens, d_in = x.shape
    n_sblocks = num_tokens // N1
    rows = b * n_sblocks
    x4 = x.reshape(rows, N1, d_in)
    grid = (rows // BS,)
    out = pl.pallas_call(
        _tc_body,
        grid=grid,
        in_specs=[
            pl.BlockSpec((BS, N1, d_in), lambda j: (j, 0, 0)),
            pl.BlockSpec((N1, D_HALF), lambda j: (0, 0)),
            pl.BlockSpec((BS, D_HALF), lambda j: (j % (n_sblocks // BS), 0)),
        ],
        out_specs=pl.BlockSpec((BS, N1, d_in), lambda j: (j, 0, 0)),
        out_shape=jax.ShapeDtypeStruct((rows, N1, d_in), x.dtype),
    )(x4, params1, params2)
    return out.reshape(b, num_tokens, d_in)
